# Initial kernel scaffold; baseline (speedup 1.0000x reference)
import jax
import jax.numpy as jnp
from jax.experimental import pallas as pl


def kernel(x, edge_index, W1, b1, W2, b2):
    raise NotImplementedError("write your pallas kernel here")


# speedup vs baseline: 9.4219x; 9.4219x over previous
import functools

import jax
import jax.numpy as jnp
from jax import lax
from jax.experimental import pallas as pl
from jax.experimental.pallas import tpu as pltpu
from jax.experimental.pallas import tpu_sc as plsc

D = 128
NP = 10240
NC = 2
NS = 16
NTILES = NC * NS
CHUNK = 128
ROWS_PT = NP // NS
RB = 1024
GRID = NP // RB


@functools.lru_cache(maxsize=None)
def _build_sc_kernels(e_pad: int):
    ept = e_pad // NTILES
    nch = ept // CHUNK
    _mesh = plsc.VectorSubcoreMesh(
        core_axis_name="c", subcore_axis_name="s",
        num_cores=NC, num_subcores=NS)

    @functools.partial(
        pl.kernel,
        out_type=jax.ShapeDtypeStruct((NC, NP), jnp.float32),
        mesh=_mesh,
        scratch_types=[
            pltpu.VMEM((CHUNK,), jnp.int32),
            pltpu.VMEM((CHUNK,), jnp.float32),
            pltpu.VMEM_SHARED((NP,), jnp.float32),
        ],
    )
    def degree_kernel(dst_hbm, ones_hbm, zeros_hbm, out_hbm, idx_v, ones_v, acc):
        cid = lax.axis_index("c")
        sid = lax.axis_index("s")
        wid = sid * NC + cid
        base = wid * ept
        pltpu.sync_copy(ones_hbm, ones_v)
        pltpu.sync_copy(zeros_hbm.at[pl.ds(sid * ROWS_PT, ROWS_PT)],
                        acc.at[pl.ds(sid * ROWS_PT, ROWS_PT)])
        plsc.subcore_barrier()

        def body(c, carry):
            pltpu.sync_copy(dst_hbm.at[pl.ds(base + c * CHUNK, CHUNK)], idx_v)
            pltpu.sync_copy(ones_v, acc.at[idx_v], add=True)
            return carry

        lax.fori_loop(0, nch, body, 0)
        plsc.subcore_barrier()
        pltpu.sync_copy(acc.at[pl.ds(sid * ROWS_PT, ROWS_PT)],
                        out_hbm.at[cid, pl.ds(sid * ROWS_PT, ROWS_PT)])

    @functools.partial(
        pl.kernel,
        out_type=jax.ShapeDtypeStruct((NC, NP, D), jnp.float32),
        mesh=_mesh,
        scratch_types=[
            pltpu.VMEM((CHUNK,), jnp.int32),
            pltpu.VMEM((CHUNK,), jnp.int32),
            pltpu.VMEM((CHUNK,), jnp.int32),
            pltpu.VMEM((CHUNK,), jnp.int32),
            pltpu.VMEM((CHUNK, D), jnp.float32),
            pltpu.VMEM((CHUNK, D), jnp.float32),
            pltpu.VMEM_SHARED((NP, D), jnp.float32),
            pltpu.SemaphoreType.DMA,
            pltpu.SemaphoreType.DMA,
        ],
    )
    def agg_kernel(h_hbm, src_hbm, dst_hbm, zrow_hbm, out_hbm,
                   s0, s1, d0, d1, r0, r1, acc, g0, g1):
        cid = lax.axis_index("c")
        sid = lax.axis_index("s")
        wid = sid * NC + cid
        base = wid * ept

        pltpu.sync_copy(zrow_hbm, r0)
        for k in range(ROWS_PT // CHUNK):
            pltpu.sync_copy(r0, acc.at[pl.ds(sid * ROWS_PT + k * CHUNK, CHUNK)])
        plsc.subcore_barrier()

        pltpu.sync_copy(src_hbm.at[pl.ds(base, CHUNK)], s0)
        pltpu.sync_copy(dst_hbm.at[pl.ds(base, CHUNK)], d0)
        pltpu.async_copy(h_hbm.at[s0], r0, g0)

        def body(g, carry):
            c1 = base + (2 * g + 1) * CHUNK
            pltpu.sync_copy(src_hbm.at[pl.ds(c1, CHUNK)], s1)
            pltpu.sync_copy(dst_hbm.at[pl.ds(c1, CHUNK)], d1)
            pltpu.async_copy(h_hbm.at[s1], r1, g1)
            pltpu.make_async_copy(h_hbm.at[s0], r0, g0).wait()
            pltpu.sync_copy(r0, acc.at[d0], add=True)

            @pl.when(g + 1 < nch // 2)
            def _():
                c2 = c1 + CHUNK
                pltpu.sync_copy(src_hbm.at[pl.ds(c2, CHUNK)], s0)
                pltpu.sync_copy(dst_hbm.at[pl.ds(c2, CHUNK)], d0)
                pltpu.async_copy(h_hbm.at[s0], r0, g0)

            pltpu.make_async_copy(h_hbm.at[s1], r1, g1).wait()
            pltpu.sync_copy(r1, acc.at[d1], add=True)
            return carry

        lax.fori_loop(0, nch // 2, body, 0)
        plsc.subcore_barrier()
        pltpu.sync_copy(acc.at[pl.ds(sid * ROWS_PT, ROWS_PT)],
                        out_hbm.at[cid, pl.ds(sid * ROWS_PT, ROWS_PT)])

    return degree_kernel, agg_kernel


def _mm1_body(x_ref, deg_ref, w_ref, o_ref):
    dinv = lax.rsqrt(deg_ref[...] + 1.0)
    o_ref[...] = jnp.dot(x_ref[...] * dinv, w_ref[...],
                         preferred_element_type=jnp.float32)


def _mid_body(acc_ref, h_ref, deg_ref, b_ref, w_ref, o_ref):
    dinv = lax.rsqrt(deg_ref[...] + 1.0)
    t = dinv * (acc_ref[0] + acc_ref[1] + h_ref[...]) + b_ref[...]
    r = jnp.maximum(t, 0.0)
    o_ref[...] = jnp.dot(r * dinv, w_ref[...],
                         preferred_element_type=jnp.float32)


def _fin_body(acc_ref, h_ref, deg_ref, b_ref, o_ref):
    dinv = lax.rsqrt(deg_ref[...] + 1.0)
    o_ref[...] = dinv * (acc_ref[0] + acc_ref[1] + h_ref[...]) + b_ref[...]


_row = pl.BlockSpec((RB, D), lambda i: (i, 0))
_col = pl.BlockSpec((RB, 1), lambda i: (i, 0))
_wsp = pl.BlockSpec((D, D), lambda i: (0, 0))
_bsp = pl.BlockSpec((1, D), lambda i: (0, 0))
_asp = pl.BlockSpec((NC, RB, D), lambda i: (0, i, 0))
_out = jax.ShapeDtypeStruct((NP, D), jnp.float32)

_mm1 = pl.pallas_call(_mm1_body, grid=(GRID,),
                      in_specs=[_row, _col, _wsp],
                      out_specs=_row, out_shape=_out)
_mid = pl.pallas_call(_mid_body, grid=(GRID,),
                      in_specs=[_asp, _row, _col, _bsp, _wsp],
                      out_specs=_row, out_shape=_out)
_fin = pl.pallas_call(_fin_body, grid=(GRID,),
                      in_specs=[_asp, _row, _col, _bsp],
                      out_specs=_row, out_shape=_out)


def kernel(x, edge_index, W1, b1, W2, b2):
    n = x.shape[0]
    e = edge_index.shape[1]
    grain = NTILES * CHUNK * 2
    e_pad = ((e + grain - 1) // grain) * grain
    degree_kernel, agg_kernel = _build_sc_kernels(e_pad)

    src = jnp.concatenate(
        [edge_index[0].astype(jnp.int32),
         jnp.zeros((e_pad - e,), jnp.int32)])
    dst = jnp.concatenate(
        [edge_index[1].astype(jnp.int32),
         jnp.full((e_pad - e,), n, jnp.int32)])
    x_p = jnp.pad(x.astype(jnp.float32), ((0, NP - n), (0, 0)))
    ones_c = jnp.ones((CHUNK,), jnp.float32)
    zeros_n = jnp.zeros((NP,), jnp.float32)
    zrow = jnp.zeros((CHUNK, D), jnp.float32)
    b1r = b1.reshape(1, D).astype(jnp.float32)
    b2r = b2.reshape(1, D).astype(jnp.float32)

    degp = degree_kernel(dst, ones_c, zeros_n)
    deg = (degp[0] + degp[1]).reshape(NP, 1)

    h1p = _mm1(x_p, deg, W1)
    acc1 = agg_kernel(h1p, src, dst, zrow)
    h2p = _mid(acc1, h1p, deg, b1r, W2)
    acc2 = agg_kernel(h2p, src, dst, zrow)
    outp = _fin(acc2, h2p, deg, b2r)
    return outp[:n]


# speedup vs baseline: 9.8671x; 1.0473x over previous
import functools

import jax
import jax.numpy as jnp
from jax import lax
from jax.experimental import pallas as pl
from jax.experimental.pallas import tpu as pltpu
from jax.experimental.pallas import tpu_sc as plsc

D = 128
NP = 10240
NC = 2
NS = 16
NTILES = NC * NS
CHUNK = 128
IB = 16
ROWS_PT = NP // NS
RB = 1024
GRID = NP // RB


@functools.lru_cache(maxsize=None)
def _build_sc_kernels(e_pad: int):
    ept = e_pad // NTILES
    nch = ept // CHUNK
    _mesh = plsc.VectorSubcoreMesh(
        core_axis_name="c", subcore_axis_name="s",
        num_cores=NC, num_subcores=NS)

    @functools.partial(
        pl.kernel,
        out_type=jax.ShapeDtypeStruct((NC, NP), jnp.float32),
        mesh=_mesh,
        scratch_types=[
            pltpu.VMEM((nch, CHUNK), jnp.int32),
            pltpu.VMEM((CHUNK,), jnp.float32),
            pltpu.VMEM((NS, ROWS_PT), jnp.float32),
            pltpu.VMEM((NS, ROWS_PT), jnp.float32),
            pltpu.VMEM((ROWS_PT,), jnp.float32),
            pltpu.VMEM_SHARED((NS * NP,), jnp.float32),
            pltpu.VMEM_SHARED((NS * NP,), jnp.float32),
            pltpu.SemaphoreType.DMA,
            pltpu.SemaphoreType.DMA,
        ],
    )
    def degree_kernel(dst_hbm, ones_hbm, zeros_hbm, out_hbm,
                      didx, ones_v, ca, cb, colsum, h0, h1, s0, s1):
        cid = lax.axis_index("c")
        sid = lax.axis_index("s")
        wid = sid * NC + cid
        hists = [h0, h1]
        sems = [s0, s1]
        pltpu.sync_copy(dst_hbm.at[wid], didx)
        pltpu.sync_copy(ones_hbm, ones_v)
        pltpu.sync_copy(zeros_hbm, h0.at[pl.ds(sid * NP, NP)])
        pltpu.sync_copy(zeros_hbm, h1.at[pl.ds(sid * NP, NP)])

        def body(g, carry):
            for b in range(2):
                c = 2 * g + b

                @pl.when(c >= 2)
                def _():
                    pltpu.make_async_copy(
                        ones_v, hists[b].at[didx.at[c - 2]], sems[b]).wait()

                pltpu.async_copy(ones_v, hists[b].at[didx.at[c]],
                                 sems[b], add=True)
            return carry

        lax.fori_loop(0, nch // 2, body, 0)
        pltpu.make_async_copy(ones_v, h0.at[didx.at[nch - 2]], s0).wait()
        pltpu.make_async_copy(ones_v, h1.at[didx.at[nch - 1]], s1).wait()
        plsc.subcore_barrier()

        for p in range(NS):
            pltpu.sync_copy(
                h0.at[pl.ds(p * NP + sid * ROWS_PT, ROWS_PT)], ca.at[p])
            pltpu.sync_copy(
                h1.at[pl.ds(p * NP + sid * ROWS_PT, ROWS_PT)], cb.at[p])

        def red(v, carry):
            s = ca[0, pl.ds(v * 16, 16)] + cb[0, pl.ds(v * 16, 16)]
            for p in range(1, NS):
                s = s + ca[p, pl.ds(v * 16, 16)] + cb[p, pl.ds(v * 16, 16)]
            colsum[pl.ds(v * 16, 16)] = s
            return carry

        lax.fori_loop(0, ROWS_PT // 16, red, 0)
        pltpu.sync_copy(colsum,
                        out_hbm.at[cid, pl.ds(sid * ROWS_PT, ROWS_PT)])

    nblk = nch // IB

    @functools.partial(
        pl.kernel,
        out_type=jax.ShapeDtypeStruct((NC, NP, D), jnp.float32),
        mesh=_mesh,
        scratch_types=[
            pltpu.VMEM((IB, CHUNK), jnp.int32),
            pltpu.VMEM((IB, CHUNK), jnp.int32),
            pltpu.VMEM_SHARED((NP, D), jnp.float32),
            pltpu.VMEM((CHUNK, D), jnp.float32),
            pltpu.VMEM((CHUNK, D), jnp.float32),
            pltpu.SemaphoreType.DMA,
            pltpu.SemaphoreType.DMA,
            pltpu.SemaphoreType.DMA,
            pltpu.SemaphoreType.DMA,
        ],
    )
    def agg_kernel(h_hbm, src_hbm, dst_hbm, zrow_hbm, out_hbm,
                   sidx, didx, acc, ra, rb, g0, g1, s0, s1):
        rows = [ra, rb]
        gsem = [g0, g1]
        ssem = [s0, s1]
        cid = lax.axis_index("c")
        sid = lax.axis_index("s")
        wid = sid * NC + cid

        pltpu.sync_copy(zrow_hbm, rows[0])
        for k in range(ROWS_PT // CHUNK):
            pltpu.sync_copy(rows[0],
                            acc.at[pl.ds(sid * ROWS_PT + k * CHUNK, CHUNK)])
        plsc.subcore_barrier()

        def gather(j, b):
            pltpu.async_copy(h_hbm.at[sidx.at[j]], rows[b], gsem[b])

        def gwait(j, b):
            pltpu.make_async_copy(h_hbm.at[sidx.at[j]], rows[b],
                                  gsem[b]).wait()

        def sstart(j, b):
            pltpu.async_copy(rows[b], acc.at[didx.at[j]], ssem[b], add=True)

        def swait(j, b):
            pltpu.make_async_copy(rows[b], acc.at[didx.at[j]], ssem[b]).wait()

        def blk(k, carry):
            pltpu.sync_copy(src_hbm.at[wid, pl.ds(k * IB, IB)], sidx)
            pltpu.sync_copy(dst_hbm.at[wid, pl.ds(k * IB, IB)], didx)
            gather(0, 0)
            gather(1, 1)

            def pair(g, carry2):
                for b in range(2):
                    j = 2 * g + b
                    gwait(j, b)
                    sstart(j, b)
                    @pl.when(j >= 1)
                    def _():
                        swait(j - 1, 1 - b)

                    @pl.when((j >= 1) & (j + 1 < IB))
                    def _():
                        gather(j + 1, 1 - b)
                return carry2

            lax.fori_loop(0, IB // 2, pair, 0)
            swait(IB - 1, (IB - 1) % 2)
            return carry

        lax.fori_loop(0, nblk, blk, 0)
        plsc.subcore_barrier()
        pltpu.sync_copy(acc.at[pl.ds(sid * ROWS_PT, ROWS_PT)],
                        out_hbm.at[cid, pl.ds(sid * ROWS_PT, ROWS_PT)])

    return degree_kernel, agg_kernel


def _mm1_body(x_ref, deg_ref, w_ref, o_ref):
    dinv = lax.rsqrt(deg_ref[...] + 1.0)
    o_ref[...] = jnp.dot(x_ref[...] * dinv, w_ref[...],
                         preferred_element_type=jnp.float32)


def _mid_body(acc_ref, h_ref, deg_ref, b_ref, w_ref, o_ref):
    dinv = lax.rsqrt(deg_ref[...] + 1.0)
    t = dinv * (acc_ref[0] + acc_ref[1] + h_ref[...]) + b_ref[...]
    r = jnp.maximum(t, 0.0)
    o_ref[...] = jnp.dot(r * dinv, w_ref[...],
                         preferred_element_type=jnp.float32)


def _fin_body(acc_ref, h_ref, deg_ref, b_ref, o_ref):
    dinv = lax.rsqrt(deg_ref[...] + 1.0)
    o_ref[...] = dinv * (acc_ref[0] + acc_ref[1] + h_ref[...]) + b_ref[...]


_row = pl.BlockSpec((RB, D), lambda i: (i, 0))
_col = pl.BlockSpec((RB, 1), lambda i: (i, 0))
_wsp = pl.BlockSpec((D, D), lambda i: (0, 0))
_bsp = pl.BlockSpec((1, D), lambda i: (0, 0))
_asp = pl.BlockSpec((NC, RB, D), lambda i: (0, i, 0))
_out = jax.ShapeDtypeStruct((NP, D), jnp.float32)

_mm1 = pl.pallas_call(_mm1_body, grid=(GRID,),
                      in_specs=[_row, _col, _wsp],
                      out_specs=_row, out_shape=_out)
_mid = pl.pallas_call(_mid_body, grid=(GRID,),
                      in_specs=[_asp, _row, _col, _bsp, _wsp],
                      out_specs=_row, out_shape=_out)
_fin = pl.pallas_call(_fin_body, grid=(GRID,),
                      in_specs=[_asp, _row, _col, _bsp],
                      out_specs=_row, out_shape=_out)


def kernel(x, edge_index, W1, b1, W2, b2):
    n = x.shape[0]
    e = edge_index.shape[1]
    grain = NTILES * CHUNK * IB
    e_pad = ((e + grain - 1) // grain) * grain
    degree_kernel, agg_kernel = _build_sc_kernels(e_pad)

    ept = e_pad // NTILES
    nch = ept // CHUNK
    src = jnp.concatenate(
        [edge_index[0].astype(jnp.int32),
         jnp.zeros((e_pad - e,), jnp.int32)]).reshape(NTILES, nch, CHUNK)
    dst = jnp.concatenate(
        [edge_index[1].astype(jnp.int32),
         n + jnp.arange(e_pad - e, dtype=jnp.int32) % (NP - n)]
    ).reshape(NTILES, nch, CHUNK)
    x_p = jnp.pad(x.astype(jnp.float32), ((0, NP - n), (0, 0)))
    ones_c = jnp.ones((CHUNK,), jnp.float32)
    zeros_n = jnp.zeros((NP,), jnp.float32)
    zrow = jnp.zeros((CHUNK, D), jnp.float32)
    b1r = b1.reshape(1, D).astype(jnp.float32)
    b2r = b2.reshape(1, D).astype(jnp.float32)

    sid_of_tile = (jnp.arange(NTILES, dtype=jnp.int32) // NC)
    dst_deg = dst + (sid_of_tile * NP)[:, None, None]
    degp = degree_kernel(dst_deg, ones_c, zeros_n)
    deg = (degp[0] + degp[1]).reshape(NP, 1)

    h1p = _mm1(x_p, deg, W1)
    acc1 = agg_kernel(h1p, src, dst, zrow)
    h2p = _mid(acc1, h1p, deg, b1r, W2)
    acc2 = agg_kernel(h2p, src, dst, zrow)
    outp = _fin(acc2, h2p, deg, b2r)
    return outp[:n]


# speedup vs baseline: 11.1293x; 1.1279x over previous
import functools

import jax
import jax.numpy as jnp
from jax import lax
from jax.experimental import pallas as pl
from jax.experimental.pallas import tpu as pltpu
from jax.experimental.pallas import tpu_sc as plsc

D = 128
NP = 10240
NC = 2
NS = 16
NTILES = NC * NS
CHUNK = 80
IB = 16
NB = 4
ROWS_PT = NP // NS
RB = 1024
GRID = NP // RB


@functools.lru_cache(maxsize=None)
def _build_sc_kernels(e_pad: int):
    ept = e_pad // NTILES
    nch = ept // CHUNK
    _mesh = plsc.VectorSubcoreMesh(
        core_axis_name="c", subcore_axis_name="s",
        num_cores=NC, num_subcores=NS)

    @functools.partial(
        pl.kernel,
        out_type=jax.ShapeDtypeStruct((NC, NP), jnp.float32),
        mesh=_mesh,
        scratch_types=[
            pltpu.VMEM((nch, CHUNK), jnp.int32),
            pltpu.VMEM((CHUNK,), jnp.float32),
            pltpu.VMEM((NS, ROWS_PT), jnp.float32),
            pltpu.VMEM((NS, ROWS_PT), jnp.float32),
            pltpu.VMEM((ROWS_PT,), jnp.float32),
            pltpu.VMEM_SHARED((NS * NP,), jnp.float32),
            pltpu.VMEM_SHARED((NS * NP,), jnp.float32),
            pltpu.SemaphoreType.DMA,
            pltpu.SemaphoreType.DMA,
        ],
    )
    def degree_kernel(dst_hbm, ones_hbm, zeros_hbm, out_hbm,
                      didx, ones_v, ca, cb, colsum, h0, h1, s0, s1):
        cid = lax.axis_index("c")
        sid = lax.axis_index("s")
        wid = sid * NC + cid
        hists = [h0, h1]
        sems = [s0, s1]
        pltpu.sync_copy(dst_hbm.at[wid], didx)
        pltpu.sync_copy(ones_hbm, ones_v)
        pltpu.sync_copy(zeros_hbm, h0.at[pl.ds(sid * NP, NP)])
        pltpu.sync_copy(zeros_hbm, h1.at[pl.ds(sid * NP, NP)])

        def body(g, carry):
            for b in range(2):
                c = 2 * g + b

                @pl.when(c >= 2)
                def _():
                    pltpu.make_async_copy(
                        ones_v, hists[b].at[didx.at[c - 2]], sems[b]).wait()

                pltpu.async_copy(ones_v, hists[b].at[didx.at[c]],
                                 sems[b], add=True)
            return carry

        lax.fori_loop(0, nch // 2, body, 0)
        pltpu.make_async_copy(ones_v, h0.at[didx.at[nch - 2]], s0).wait()
        pltpu.make_async_copy(ones_v, h1.at[didx.at[nch - 1]], s1).wait()
        plsc.subcore_barrier()

        for p in range(NS):
            pltpu.sync_copy(
                h0.at[pl.ds(p * NP + sid * ROWS_PT, ROWS_PT)], ca.at[p])
            pltpu.sync_copy(
                h1.at[pl.ds(p * NP + sid * ROWS_PT, ROWS_PT)], cb.at[p])

        def red(v, carry):
            s = ca[0, pl.ds(v * 16, 16)] + cb[0, pl.ds(v * 16, 16)]
            for p in range(1, NS):
                s = s + ca[p, pl.ds(v * 16, 16)] + cb[p, pl.ds(v * 16, 16)]
            colsum[pl.ds(v * 16, 16)] = s
            return carry

        lax.fori_loop(0, ROWS_PT // 16, red, 0)
        pltpu.sync_copy(colsum,
                        out_hbm.at[cid, pl.ds(sid * ROWS_PT, ROWS_PT)])

    nblk = nch // IB

    @functools.partial(
        pl.kernel,
        out_type=jax.ShapeDtypeStruct((NC, NP, D), jnp.float32),
        mesh=_mesh,
        scratch_types=[
            pltpu.VMEM((IB, CHUNK), jnp.int32),
            pltpu.VMEM((IB, CHUNK), jnp.int32),
            pltpu.VMEM_SHARED((NP, D), jnp.float32),
            pltpu.VMEM((CHUNK, D), jnp.float32),
            pltpu.VMEM((CHUNK, D), jnp.float32),
            pltpu.VMEM((CHUNK, D), jnp.float32),
            pltpu.VMEM((CHUNK, D), jnp.float32),
            pltpu.SemaphoreType.DMA,
            pltpu.SemaphoreType.DMA,
            pltpu.SemaphoreType.DMA,
            pltpu.SemaphoreType.DMA,
            pltpu.SemaphoreType.DMA,
            pltpu.SemaphoreType.DMA,
            pltpu.SemaphoreType.DMA,
            pltpu.SemaphoreType.DMA,
        ],
    )
    def agg_kernel(h_hbm, src_hbm, dst_hbm, zrow_hbm, out_hbm,
                   sidx, didx, acc, ra, rb, rc, rd,
                   g0, g1, g2, g3, s0, s1, s2, s3):
        rows = [ra, rb, rc, rd]
        gsem = [g0, g1, g2, g3]
        ssem = [s0, s1, s2, s3]
        cid = lax.axis_index("c")
        sid = lax.axis_index("s")
        wid = sid * NC + cid

        pltpu.sync_copy(zrow_hbm, rows[0])
        for k in range(ROWS_PT // CHUNK):
            pltpu.sync_copy(rows[0],
                            acc.at[pl.ds(sid * ROWS_PT + k * CHUNK, CHUNK)])
        plsc.subcore_barrier()

        def gather(j, b):
            pltpu.async_copy(h_hbm.at[sidx.at[j]], rows[b], gsem[b])

        def gwait(j, b):
            pltpu.make_async_copy(h_hbm.at[sidx.at[j]], rows[b],
                                  gsem[b]).wait()

        def sstart(j, b):
            pltpu.async_copy(rows[b], acc.at[didx.at[j]], ssem[b], add=True)

        def swait(j, b):
            pltpu.make_async_copy(rows[b], acc.at[didx.at[j]], ssem[b]).wait()

        def blk(k, carry):
            pltpu.sync_copy(src_hbm.at[wid, pl.ds(k * IB, IB)], sidx)
            pltpu.sync_copy(dst_hbm.at[wid, pl.ds(k * IB, IB)], didx)
            gather(0, 0)
            gather(1, 1)

            def quad(g, carry2):
                for b in range(NB):
                    j = NB * g + b
                    bn = (b + 2) % NB
                    @pl.when(j >= 2)
                    def _():
                        swait(j - 2, bn)

                    @pl.when(j + 2 < IB)
                    def _():
                        gather(j + 2, bn)

                    gwait(j, b)
                    sstart(j, b)
                return carry2

            lax.fori_loop(0, IB // NB, quad, 0)
            swait(IB - 2, (IB - 2) % NB)
            swait(IB - 1, (IB - 1) % NB)
            return carry

        lax.fori_loop(0, nblk, blk, 0)
        plsc.subcore_barrier()
        pltpu.sync_copy(acc.at[pl.ds(sid * ROWS_PT, ROWS_PT)],
                        out_hbm.at[cid, pl.ds(sid * ROWS_PT, ROWS_PT)])

    return degree_kernel, agg_kernel


def _mm1_body(x_ref, deg_ref, w_ref, o_ref):
    dinv = lax.rsqrt(deg_ref[...] + 1.0)
    o_ref[...] = jnp.dot(x_ref[...] * dinv, w_ref[...],
                         preferred_element_type=jnp.float32)


def _mid_body(acc_ref, h_ref, deg_ref, b_ref, w_ref, o_ref):
    dinv = lax.rsqrt(deg_ref[...] + 1.0)
    t = dinv * (acc_ref[0] + acc_ref[1] + h_ref[...]) + b_ref[...]
    r = jnp.maximum(t, 0.0)
    o_ref[...] = jnp.dot(r * dinv, w_ref[...],
                         preferred_element_type=jnp.float32)


def _fin_body(acc_ref, h_ref, deg_ref, b_ref, o_ref):
    dinv = lax.rsqrt(deg_ref[...] + 1.0)
    o_ref[...] = dinv * (acc_ref[0] + acc_ref[1] + h_ref[...]) + b_ref[...]


_row = pl.BlockSpec((RB, D), lambda i: (i, 0))
_col = pl.BlockSpec((RB, 1), lambda i: (i, 0))
_wsp = pl.BlockSpec((D, D), lambda i: (0, 0))
_bsp = pl.BlockSpec((1, D), lambda i: (0, 0))
_asp = pl.BlockSpec((NC, RB, D), lambda i: (0, i, 0))
_out = jax.ShapeDtypeStruct((NP, D), jnp.float32)

_mm1 = pl.pallas_call(_mm1_body, grid=(GRID,),
                      in_specs=[_row, _col, _wsp],
                      out_specs=_row, out_shape=_out)
_mid = pl.pallas_call(_mid_body, grid=(GRID,),
                      in_specs=[_asp, _row, _col, _bsp, _wsp],
                      out_specs=_row, out_shape=_out)
_fin = pl.pallas_call(_fin_body, grid=(GRID,),
                      in_specs=[_asp, _row, _col, _bsp],
                      out_specs=_row, out_shape=_out)


def kernel(x, edge_index, W1, b1, W2, b2):
    n = x.shape[0]
    e = edge_index.shape[1]
    grain = NTILES * CHUNK * IB
    e_pad = ((e + grain - 1) // grain) * grain
    degree_kernel, agg_kernel = _build_sc_kernels(e_pad)

    ept = e_pad // NTILES
    nch = ept // CHUNK
    src = jnp.concatenate(
        [edge_index[0].astype(jnp.int32),
         jnp.zeros((e_pad - e,), jnp.int32)]).reshape(NTILES, nch, CHUNK)
    dst = jnp.concatenate(
        [edge_index[1].astype(jnp.int32),
         n + jnp.arange(e_pad - e, dtype=jnp.int32) % (NP - n)]
    ).reshape(NTILES, nch, CHUNK)
    x_p = jnp.pad(x.astype(jnp.float32), ((0, NP - n), (0, 0)))
    ones_c = jnp.ones((CHUNK,), jnp.float32)
    zeros_n = jnp.zeros((NP,), jnp.float32)
    zrow = jnp.zeros((CHUNK, D), jnp.float32)
    b1r = b1.reshape(1, D).astype(jnp.float32)
    b2r = b2.reshape(1, D).astype(jnp.float32)

    sid_of_tile = (jnp.arange(NTILES, dtype=jnp.int32) // NC)
    dst_deg = dst + (sid_of_tile * NP)[:, None, None]
    degp = degree_kernel(dst_deg, ones_c, zeros_n)
    deg = (degp[0] + degp[1]).reshape(NP, 1)

    h1p = _mm1(x_p, deg, W1)
    acc1 = agg_kernel(h1p, src, dst, zrow)
    h2p = _mid(acc1, h1p, deg, b1r, W2)
    acc2 = agg_kernel(h2p, src, dst, zrow)
    outp = _fin(acc2, h2p, deg, b2r)
    return outp[:n]
